# Initial kernel scaffold; baseline (speedup 1.0000x reference)
import jax
import jax.numpy as jnp
from jax.experimental import pallas as pl


def kernel(local_coordinate, flattened_index, convolved):
    raise NotImplementedError("write your pallas kernel here")


# speedup vs baseline: 2.6568x; 2.6568x over previous
import functools

import jax
import jax.numpy as jnp
from jax import lax
from jax.experimental import pallas as pl
from jax.experimental.pallas import tpu as pltpu
from jax.experimental.pallas import tpu_sc as plsc

H = 4
S = 8
P = 65536
F = 16
V = 64 * 64 * 64

NC = 2
NS = 16
PT = P // NS
HEADS_PER_CORE = H // NC
PLANE_CHUNK = V // NS


def _sc_body(lc_hbm, fi_hbm, cv_hbm, out_hbm, plane_sh, w_v, out_v, *idx_g):
    idx_refs = idx_g[:S]
    g_refs = idx_g[S:]
    cid = lax.axis_index("c")
    sid = lax.axis_index("s")

    for h2 in range(HEADS_PER_CORE):
        h = cid * HEADS_PER_CORE + h2
        for s in range(S):
            pltpu.sync_copy(fi_hbm.at[h, s, sid, :], idx_refs[s])
        pltpu.sync_copy(lc_hbm.at[h, :, sid, :], w_v)

        for f in range(F):
            row = h * F + f
            pltpu.sync_copy(
                cv_hbm.at[row, pl.ds(sid * PLANE_CHUNK, PLANE_CHUNK)],
                plane_sh.at[pl.ds(sid * PLANE_CHUNK, PLANE_CHUNK)],
            )
            plsc.subcore_barrier()

            for s in range(S):
                pltpu.sync_copy(plane_sh.at[idx_refs[s]], g_refs[s])

            def _chunk_body(j, _):
                col = j * 16
                acc = w_v[0, pl.ds(col, 16)] * g_refs[0][pl.ds(col, 16)]
                for s in range(1, S):
                    acc = acc + w_v[s, pl.ds(col, 16)] * g_refs[s][pl.ds(col, 16)]
                out_v[pl.ds(col, 16)] = acc
                return 0

            lax.fori_loop(0, PT // 16, _chunk_body, 0)

            pltpu.sync_copy(out_v, out_hbm.at[row, sid, :])
            plsc.subcore_barrier()


@jax.jit
def _slice_sc(lc, fi, cv):
    mesh = plsc.VectorSubcoreMesh(
        core_axis_name="c", subcore_axis_name="s", num_cores=NC, num_subcores=NS
    )
    run = pl.kernel(
        _sc_body,
        out_type=jax.ShapeDtypeStruct((H * F, NS, PT), jnp.float32),
        mesh=mesh,
        scratch_types=[
            pltpu.VMEM_SHARED((V,), jnp.float32),
            pltpu.VMEM((S, PT), jnp.float32),
            pltpu.VMEM((PT,), jnp.float32),
        ]
        + [pltpu.VMEM((PT,), jnp.int32) for _ in range(S)]
        + [pltpu.VMEM((PT,), jnp.float32) for _ in range(S)],
    )
    return run(lc, fi, cv)


def kernel(local_coordinate, flattened_index, convolved):
    lc = local_coordinate.reshape(H, S, NS, PT)
    fi = flattened_index.reshape(H, S, NS, PT).astype(jnp.int32)
    cv = convolved.reshape(H * F, V)
    out = _slice_sc(lc, fi, cv)
    return out.reshape(1, H * F, P)
